# baseline (speedup 1.0000x reference)
import functools

import jax
import jax.numpy as jnp
from jax import lax
from jax.experimental import pallas as pl
from jax.experimental.pallas import tpu as pltpu
from jax.experimental.pallas import tpu_sc as plsc

N = 100000
D = 128
CHUNK = 400
NCHUNK = N // CHUNK
NC, NS = 2, 16
NW = NC * NS
MAXI = -(-NCHUNK // NW)


def _body(idx_hbm, emb_hbm, out_hbm,
          idx_v0, idx_v1, rows_v0, rows_v1, sem_g, sem_o0, sem_o1):
    c = lax.axis_index("c")
    s = lax.axis_index("s")
    wid = s * NC + c

    idx_v = (idx_v0, idx_v1)
    rows_v = (rows_v0, rows_v1)
    sem_o = (sem_o0, sem_o1)
    out_copies = []

    for i in range(MAXI):
        k = wid + i * NW
        b = i % 2

        @pl.when(k < NCHUNK)
        def _process(k=k, b=b, i=i):
            if i >= 2:
                out_copies[i - 2].wait()
            pltpu.sync_copy(idx_hbm.at[k], idx_v[b])
            pltpu.async_copy(emb_hbm.at[idx_v[b]], rows_v[b], sem_g).wait()
            out_copies.append(
                pltpu.async_copy(rows_v[b], out_hbm.at[k], sem_o[b]))

    for i in range(MAXI - 3, MAXI):
        k = wid + i * NW

        @pl.when((k < NCHUNK) & (k + 2 * NW >= NCHUNK))
        def _drain(i=i):
            out_copies[i].wait()


@jax.jit
def _embed(idx2, emb_pad):
    mesh = plsc.VectorSubcoreMesh(core_axis_name="c", subcore_axis_name="s")
    f = functools.partial(
        pl.kernel,
        out_type=jax.ShapeDtypeStruct((NCHUNK, CHUNK, D), jnp.float32),
        mesh=mesh,
        scratch_types=[
            pltpu.VMEM((CHUNK,), jnp.int32),
            pltpu.VMEM((CHUNK,), jnp.int32),
            pltpu.VMEM((CHUNK, D), jnp.float32),
            pltpu.VMEM((CHUNK, D), jnp.float32),
            pltpu.SemaphoreType.DMA,
            pltpu.SemaphoreType.DMA,
            pltpu.SemaphoreType.DMA,
        ],
    )(_body)
    return f(idx2, emb_pad)


def kernel(inputs, embedding):
    idx2 = inputs.reshape(NCHUNK, CHUNK)
    emb_pad = jnp.concatenate(
        [jnp.zeros((1, D), jnp.float32), embedding], axis=0)
    out = _embed(idx2, emb_pad)
    return out.reshape(N, D)


# speedup vs baseline: 4.3001x; 4.3001x over previous
import functools

import jax
import jax.numpy as jnp
from jax import lax
from jax.experimental import pallas as pl
from jax.experimental.pallas import tpu as pltpu
from jax.experimental.pallas import tpu_sc as plsc

N = 100000
D = 128
CHUNK = 400
NCHUNK = N // CHUNK
NC, NS = 2, 16
NW = NC * NS
MAXI = -(-NCHUNK // NW)


def _body(idx_hbm, emb_hbm, out_hbm,
          table_sh, idx_v0, idx_v1, rows_v0, rows_v1, sem_g, sem_o0, sem_o1):
    c = lax.axis_index("c")
    s = lax.axis_index("s")
    wid = s * NC + c

    @pl.when(s == 0)
    def _stage():
        pltpu.sync_copy(emb_hbm, table_sh)

    plsc.subcore_barrier()

    idx_v = (idx_v0, idx_v1)
    rows_v = (rows_v0, rows_v1)
    sem_o = (sem_o0, sem_o1)
    out_copies = []

    for i in range(MAXI):
        k = wid + i * NW
        b = i % 2

        @pl.when(k < NCHUNK)
        def _process(k=k, b=b, i=i):
            if i >= 2:
                out_copies[i - 2].wait()
            pltpu.sync_copy(idx_hbm.at[k], idx_v[b])
            pltpu.async_copy(table_sh.at[idx_v[b]], rows_v[b], sem_g).wait()
            out_copies.append(
                pltpu.async_copy(rows_v[b], out_hbm.at[k], sem_o[b]))

    for i in range(MAXI - 3, MAXI):
        k = wid + i * NW

        @pl.when((k < NCHUNK) & (k + 2 * NW >= NCHUNK))
        def _drain(i=i):
            out_copies[i].wait()


@jax.jit
def _embed(idx2, emb_pad):
    mesh = plsc.VectorSubcoreMesh(core_axis_name="c", subcore_axis_name="s")
    f = functools.partial(
        pl.kernel,
        out_type=jax.ShapeDtypeStruct((NCHUNK, CHUNK, D), jnp.float32),
        mesh=mesh,
        scratch_types=[
            pltpu.VMEM_SHARED((37, D), jnp.float32),
            pltpu.VMEM((CHUNK,), jnp.int32),
            pltpu.VMEM((CHUNK,), jnp.int32),
            pltpu.VMEM((CHUNK, D), jnp.float32),
            pltpu.VMEM((CHUNK, D), jnp.float32),
            pltpu.SemaphoreType.DMA,
            pltpu.SemaphoreType.DMA,
            pltpu.SemaphoreType.DMA,
        ],
    )(_body)
    return f(idx2, emb_pad)


def kernel(inputs, embedding):
    idx2 = inputs.reshape(NCHUNK, CHUNK)
    emb_pad = jnp.concatenate(
        [jnp.zeros((1, D), jnp.float32), embedding], axis=0)
    out = _embed(idx2, emb_pad)
    return out.reshape(N, D)


# speedup vs baseline: 4.6249x; 1.0755x over previous
import functools

import jax
import jax.numpy as jnp
from jax import lax
from jax.experimental import pallas as pl
from jax.experimental.pallas import tpu as pltpu
from jax.experimental.pallas import tpu_sc as plsc

N = 100000
D = 128
CHUNK = 400
NCHUNK = N // CHUNK
NC, NS = 2, 16
NW = NC * NS
MAXI = -(-NCHUNK // NW)


def _body(idx_hbm, emb_hbm, out_hbm,
          table_sh, i0, i1, i2, i3, i4, i5, i6, i7, rows_v0, rows_v1,
          sem_i, sem_g0, sem_g1, sem_o0, sem_o1):
    idx_v = (i0, i1, i2, i3, i4, i5, i6, i7)
    c = lax.axis_index("c")
    s = lax.axis_index("s")
    wid = s * NC + c

    def guard(i):
        return wid + i * NW < NCHUNK

    idx_copies = {}
    for i in range(MAXI):
        k = wid + i * NW

        @pl.when(guard(i))
        def _fetch(i=i, k=k):
            idx_copies[i] = pltpu.async_copy(idx_hbm.at[k], idx_v[i], sem_i)

    @pl.when(s == 0)
    def _stage():
        pltpu.sync_copy(emb_hbm, table_sh)

    plsc.subcore_barrier()

    for i in range(MAXI):
        @pl.when(guard(i))
        def _drain_idx(i=i):
            idx_copies[i].wait()

    rows_v = (rows_v0, rows_v1)
    sem_g = (sem_g0, sem_g1)
    sem_o = (sem_o0, sem_o1)
    gathers = {}
    outs = {}

    @pl.when(guard(0))
    def _g0():
        gathers[0] = pltpu.async_copy(
            table_sh.at[idx_v[0]], rows_v0, sem_g0)

    for i in range(MAXI):
        k = wid + i * NW

        if i + 1 < MAXI:
            @pl.when(guard(i + 1))
            def _issue_next(i=i):
                b = (i + 1) % 2
                if i - 1 >= 0:
                    outs[i - 1].wait()
                gathers[i + 1] = pltpu.async_copy(
                    table_sh.at[idx_v[i + 1]], rows_v[b], sem_g[b])

        @pl.when(guard(i))
        def _finish(i=i, k=k):
            gathers[i].wait()
            outs[i] = pltpu.async_copy(rows_v[i % 2], out_hbm.at[k],
                                       sem_o[i % 2])

    for i in range(MAXI - 3, MAXI):
        k = wid + i * NW

        @pl.when((k < NCHUNK) & (k + 2 * NW >= NCHUNK))
        def _drain(i=i):
            outs[i].wait()


@jax.jit
def _embed(idx2, emb_pad):
    mesh = plsc.VectorSubcoreMesh(core_axis_name="c", subcore_axis_name="s")
    f = functools.partial(
        pl.kernel,
        out_type=jax.ShapeDtypeStruct((NCHUNK, CHUNK, D), jnp.float32),
        mesh=mesh,
        scratch_types=[
            pltpu.VMEM_SHARED((37, D), jnp.float32),
        ] + [pltpu.VMEM((CHUNK,), jnp.int32)] * MAXI + [
            pltpu.VMEM((CHUNK, D), jnp.float32),
            pltpu.VMEM((CHUNK, D), jnp.float32),
            pltpu.SemaphoreType.DMA,
            pltpu.SemaphoreType.DMA,
            pltpu.SemaphoreType.DMA,
            pltpu.SemaphoreType.DMA,
            pltpu.SemaphoreType.DMA,
        ],
    )(_body)
    return f(idx2, emb_pad)


def kernel(inputs, embedding):
    idx2 = inputs.reshape(NCHUNK, CHUNK)
    emb_pad = jnp.concatenate(
        [jnp.zeros((1, D), jnp.float32), embedding], axis=0)
    out = _embed(idx2, emb_pad)
    return out.reshape(N, D)


# speedup vs baseline: 4.7008x; 1.0164x over previous
import functools

import jax
import jax.numpy as jnp
from jax import lax
from jax.experimental import pallas as pl
from jax.experimental.pallas import tpu as pltpu
from jax.experimental.pallas import tpu_sc as plsc

N = 100000
D = 128
CHUNK = 400
NCHUNK = N // CHUNK
NC, NS = 2, 16
NW = NC * NS
MAXI = -(-NCHUNK // NW)


def _body(idx_hbm, emb_hbm, out_hbm,
          table_sh, i0, i1, i2, i3, i4, i5, i6, i7, rows_v0, rows_v1,
          sem_i, sem_g0, sem_g1, sem_o0, sem_o1):
    idx_v = (i0, i1, i2, i3, i4, i5, i6, i7)
    c = lax.axis_index("c")
    s = lax.axis_index("s")
    wid = s * NC + c

    def guard(i):
        return wid + i * NW < NCHUNK

    idx_copies = {}
    for i in range(MAXI):
        k = wid + i * NW

        @pl.when(guard(i))
        def _fetch(i=i, k=k):
            idx_copies[i] = pltpu.async_copy(idx_hbm.at[k], idx_v[i], sem_i)

    @pl.when(s == 0)
    def _stage():
        pltpu.sync_copy(emb_hbm, table_sh.at[pl.ds(1, 36)])

    plsc.subcore_barrier()

    for i in range(MAXI):
        @pl.when(guard(i))
        def _drain_idx(i=i):
            idx_copies[i].wait()

    rows_v = (rows_v0, rows_v1)
    sem_g = (sem_g0, sem_g1)
    sem_o = (sem_o0, sem_o1)
    gathers = {}
    outs = {}

    @pl.when(guard(0))
    def _g0():
        gathers[0] = pltpu.async_copy(
            table_sh.at[idx_v[0]], rows_v0, sem_g0)

    for i in range(MAXI):
        k = wid + i * NW

        if i + 1 < MAXI:
            @pl.when(guard(i + 1))
            def _issue_next(i=i):
                b = (i + 1) % 2
                if i - 1 >= 0:
                    outs[i - 1].wait()
                gathers[i + 1] = pltpu.async_copy(
                    table_sh.at[idx_v[i + 1]], rows_v[b], sem_g[b])

        @pl.when(guard(i))
        def _finish(i=i, k=k):
            gathers[i].wait()
            outs[i] = pltpu.async_copy(rows_v[i % 2], out_hbm.at[k],
                                       sem_o[i % 2])

    for i in range(MAXI - 3, MAXI):
        k = wid + i * NW

        @pl.when((k < NCHUNK) & (k + 2 * NW >= NCHUNK))
        def _drain(i=i):
            outs[i].wait()


@jax.jit
def _embed(idx, embedding):
    mesh = plsc.VectorSubcoreMesh(core_axis_name="c", subcore_axis_name="s")
    f = functools.partial(
        pl.kernel,
        out_type=jax.ShapeDtypeStruct((NCHUNK, CHUNK, D), jnp.float32),
        mesh=mesh,
        scratch_types=[
            pltpu.VMEM_SHARED((37, D), jnp.float32),
        ] + [pltpu.VMEM((CHUNK,), jnp.int32)] * MAXI + [
            pltpu.VMEM((CHUNK, D), jnp.float32),
            pltpu.VMEM((CHUNK, D), jnp.float32),
            pltpu.SemaphoreType.DMA,
            pltpu.SemaphoreType.DMA,
            pltpu.SemaphoreType.DMA,
            pltpu.SemaphoreType.DMA,
            pltpu.SemaphoreType.DMA,
        ],
    )(_body)
    return f(idx, embedding)


def kernel(inputs, embedding):
    out = _embed(inputs.reshape(NCHUNK, CHUNK), embedding)
    return out.reshape(N, D)
